# baseline (speedup 1.0000x reference)
import functools

import jax
import jax.numpy as jnp
from jax import lax
from jax.experimental import pallas as pl
from jax.experimental.pallas import tpu as pltpu
from jax.experimental.pallas import tpu_sc as plsc

VOCAB = 8192
ZC = 32
N_TOK = 8192
TB = 256
K1_GRID = N_TOK // TB
BLK_PER_BATCH = 1024 // TB
NW = 32
BPW = N_TOK // NW
ROWS_PER_WIDE = 128 // ZC
MARGIN = 1.0 * (N_TOK * ZC / ZC) / VOCAB * 0.08


def _k1_body(z_ref, emb_ref, idx_ref, embn_ref, e2_ref):
    i = pl.program_id(0)

    @pl.when(i == 0)
    def _():
        e = emb_ref[...]
        n = jnp.sqrt(jnp.sum(e * e, axis=1, keepdims=True))
        en = e / jnp.maximum(n, 1e-12)
        embn_ref[...] = (-2.0 * en).astype(jnp.bfloat16)
        e2_ref[...] = jnp.sum(en * en, axis=1, keepdims=True)

    zb = z_ref[0]
    zn = zb / jnp.maximum(jnp.sqrt(jnp.sum(zb * zb, axis=0, keepdims=True)), 1e-12)
    s2 = lax.dot_general(embn_ref[...], zn.astype(jnp.bfloat16),
                         (((1,), (0,)), ((), ())),
                         preferred_element_type=jnp.float32)
    z2 = jnp.sum(zn * zn, axis=0, keepdims=True)
    d = (z2 + e2_ref[...]) + s2
    mn = jnp.min(d, axis=0, keepdims=True)
    rows = lax.broadcasted_iota(jnp.int32, d.shape, 0)
    cand = jnp.where(d == mn, rows, jnp.int32(2**30))
    idx_ref[0, 0, :] = jnp.min(cand, axis=0)


def _k1(z3, emb):
    return pl.pallas_call(
        _k1_body,
        grid=(K1_GRID,),
        in_specs=[
            pl.BlockSpec((1, ZC, TB),
                         lambda i: (i // BLK_PER_BATCH, 0, i % BLK_PER_BATCH)),
            pl.BlockSpec((VOCAB, ZC), lambda i: (0, 0)),
        ],
        out_specs=pl.BlockSpec((1, 1, TB), lambda i: (i, 0, 0)),
        out_shape=jax.ShapeDtypeStruct((K1_GRID, 1, TB), jnp.int32),
        scratch_shapes=[
            pltpu.VMEM((VOCAB, ZC), jnp.bfloat16),
            pltpu.VMEM((VOCAB, 1), jnp.float32),
        ],
    )(z3, emb)


def _k2_body(idx_hbm, emb_hbm, zqw_hbm, hist_hbm,
             idx_v, ridx_v, rows_v, hist_v, sem):
    wid = lax.axis_index("s") * 2 + lax.axis_index("c")
    base = wid * BPW
    pltpu.sync_copy(idx_hbm.at[pl.ds(base, BPW)], idx_v)

    def _shift(j, c):
        ridx_v[pl.ds(j * 16, 16)] = lax.shift_right_logical(
            idx_v[pl.ds(j * 16, 16)], 2)
        return c
    lax.fori_loop(0, BPW // 16, _shift, 0)

    for k in range(BPW // 128):
        pltpu.async_copy(emb_hbm.at[ridx_v.at[pl.ds(k * 128, 128)]],
                         rows_v.at[pl.ds(k * 128, 128)], sem).wait()
    pltpu.sync_copy(rows_v, zqw_hbm.at[pl.ds(base, BPW)])

    def _zero(j, c):
        hist_v[pl.ds(j * 16, 16)] = jnp.zeros((16,), jnp.float32)
        return c
    lax.fori_loop(0, VOCAB // 16, _zero, 0)
    ones = jnp.ones((16,), jnp.float32)

    def _hist(j, c):
        iv = idx_v[pl.ds(j * 16, 16)]
        plsc.addupdate_scatter(hist_v, [iv], ones)
        return c
    lax.fori_loop(0, BPW // 16, _hist, 0)
    pltpu.sync_copy(hist_v, hist_hbm.at[wid])


def _k2(idx, emb128):
    mesh = plsc.VectorSubcoreMesh(core_axis_name="c", subcore_axis_name="s")
    f = functools.partial(
        pl.kernel,
        out_type=[
            jax.ShapeDtypeStruct((N_TOK, 128), jnp.float32),
            jax.ShapeDtypeStruct((NW, VOCAB), jnp.float32),
        ],
        mesh=mesh,
        scratch_types=[
            pltpu.VMEM((BPW,), jnp.int32),
            pltpu.VMEM((BPW,), jnp.int32),
            pltpu.VMEM((BPW, 128), jnp.float32),
            pltpu.VMEM((VOCAB,), jnp.float32),
            pltpu.SemaphoreType.DMA,
        ],
        compiler_params=pltpu.CompilerParams(needs_layout_passes=False),
    )(_k2_body)
    return f(idx, emb128)


def _k3_body(z_ref, zqwt_ref, rem_ref, hist_ref,
             o_ref, us_ref, vq_ref, cl_ref, acc_ref):
    i = pl.program_id(0)
    zb = z_ref[0]
    zn = zb / jnp.maximum(jnp.sqrt(jnp.sum(zb * zb, axis=0, keepdims=True)), 1e-12)
    rem = rem_ref[...] & 3
    w = zqwt_ref[...]
    q = w[0:ZC, :]
    for r in range(1, ROWS_PER_WIDE):
        q = jnp.where(rem == r, w[r * ZC:(r + 1) * ZC, :], q)
    qn = q / jnp.maximum(jnp.sqrt(jnp.sum(q * q, axis=0, keepdims=True)), 1e-12)
    diff = qn - zn
    o_ref[0] = zn + diff
    part = jnp.sum(diff * diff)

    @pl.when(i == 0)
    def _():
        acc_ref[0] = part

    @pl.when(i > 0)
    def _():
        acc_ref[0] = acc_ref[0] + part

    @pl.when(i == pl.num_programs(0) - 1)
    def _():
        m = acc_ref[0] / jnp.float32(N_TOK * ZC)
        vq_ref[...] = m.reshape(1, 1)
        cl_ref[...] = (jnp.float32(0.25) * m).reshape(1, 1)
        hit = jnp.sum(hist_ref[...], axis=0)
        used = (hit >= MARGIN).astype(jnp.float32)
        us_ref[...] = (jnp.mean(used) * 100.0).reshape(1, 1)


def _k3(z3, zqwt, idx2, hist):
    return pl.pallas_call(
        _k3_body,
        grid=(8,),
        in_specs=[
            pl.BlockSpec((1, ZC, 1024), lambda i: (i, 0, 0)),
            pl.BlockSpec((128, 1024), lambda i: (0, i)),
            pl.BlockSpec((1, 1024), lambda i: (0, i)),
            pl.BlockSpec((NW, VOCAB), lambda i: (0, 0)),
        ],
        out_specs=[
            pl.BlockSpec((1, ZC, 1024), lambda i: (i, 0, 0)),
            pl.BlockSpec((1, 1), lambda i: (0, 0)),
            pl.BlockSpec((1, 1), lambda i: (0, 0)),
            pl.BlockSpec((1, 1), lambda i: (0, 0)),
        ],
        out_shape=[
            jax.ShapeDtypeStruct((8, ZC, 1024), jnp.float32),
            jax.ShapeDtypeStruct((1, 1), jnp.float32),
            jax.ShapeDtypeStruct((1, 1), jnp.float32),
            jax.ShapeDtypeStruct((1, 1), jnp.float32),
        ],
        scratch_shapes=[pltpu.SMEM((1,), jnp.float32)],
    )(z3, zqwt, idx2, hist)


def kernel(z, embedding):
    z3 = z.reshape(8, ZC, 1024)
    idx = _k1(z3, embedding).reshape(N_TOK)
    emb128 = embedding.reshape(VOCAB // ROWS_PER_WIDE, 128)
    zqw, hist = _k2(idx, emb128)
    zqwt = zqw.T
    zq_out, usage, vq, cl = _k3(z3, zqwt, idx.reshape(1, N_TOK), hist)
    return (zq_out.reshape(8, ZC, 32, 32), usage[0, 0], vq[0, 0], cl[0, 0])


# speedup vs baseline: 1.7046x; 1.7046x over previous
import functools

import jax
import jax.numpy as jnp
from jax import lax
from jax.experimental import pallas as pl
from jax.experimental.pallas import tpu as pltpu
from jax.experimental.pallas import tpu_sc as plsc

VOCAB = 8192
ZC = 32
N_TOK = 8192
TB = 1024
K1_GRID = N_TOK // TB
BLK_PER_BATCH = 1024 // TB
NW = 32
BPW = N_TOK // NW
ROWS_PER_WIDE = 128 // ZC
MARGIN = 1.0 * (N_TOK * ZC / ZC) / VOCAB * 0.08


def _k1_body(z_ref, emb_ref, idx_ref, embn_ref, e2_ref):
    i = pl.program_id(0)

    @pl.when(i == 0)
    def _():
        e = emb_ref[...]
        n = jnp.sqrt(jnp.sum(e * e, axis=1, keepdims=True))
        en = e / jnp.maximum(n, 1e-12)
        embn_ref[...] = (-2.0 * en).astype(jnp.bfloat16)
        e2_ref[...] = jnp.sum(en * en, axis=1, keepdims=True)

    zb = z_ref[0]
    zn = zb / jnp.maximum(jnp.sqrt(jnp.sum(zb * zb, axis=0, keepdims=True)), 1e-12)
    s2 = lax.dot_general(embn_ref[...], zn.astype(jnp.bfloat16),
                         (((1,), (0,)), ((), ())),
                         preferred_element_type=jnp.float32)
    d = e2_ref[...] + s2
    idx_ref[0, 0, :] = jnp.argmin(d, axis=0).astype(jnp.int32)


def _k1(z3, emb):
    return pl.pallas_call(
        _k1_body,
        grid=(K1_GRID,),
        in_specs=[
            pl.BlockSpec((1, ZC, TB),
                         lambda i: (i // BLK_PER_BATCH, 0, i % BLK_PER_BATCH)),
            pl.BlockSpec((VOCAB, ZC), lambda i: (0, 0)),
        ],
        out_specs=pl.BlockSpec((1, 1, TB), lambda i: (i, 0, 0)),
        out_shape=jax.ShapeDtypeStruct((K1_GRID, 1, TB), jnp.int32),
        scratch_shapes=[
            pltpu.VMEM((VOCAB, ZC), jnp.bfloat16),
            pltpu.VMEM((VOCAB, 1), jnp.float32),
        ],
    )(z3, emb)


def _k2_body(idx_hbm, emb_hbm, zqw_hbm, hist_hbm,
             idx_v, ridx_v, rows_v, hist_v, sem):
    wid = lax.axis_index("s") * 2 + lax.axis_index("c")
    base = wid * BPW
    pltpu.sync_copy(idx_hbm.at[pl.ds(base, BPW)], idx_v)

    def _shift(j, c):
        ridx_v[pl.ds(j * 16, 16)] = lax.shift_right_logical(
            idx_v[pl.ds(j * 16, 16)], 2)
        return c
    lax.fori_loop(0, BPW // 16, _shift, 0)

    for k in range(BPW // 128):
        pltpu.async_copy(emb_hbm.at[ridx_v.at[pl.ds(k * 128, 128)]],
                         rows_v.at[pl.ds(k * 128, 128)], sem).wait()
    pltpu.sync_copy(rows_v, zqw_hbm.at[pl.ds(base, BPW)])

    def _zero(j, c):
        hist_v[pl.ds(j * 16, 16)] = jnp.zeros((16,), jnp.float32)
        return c
    lax.fori_loop(0, VOCAB // 16, _zero, 0)
    ones = jnp.ones((16,), jnp.float32)

    def _hist(j, c):
        iv = idx_v[pl.ds(j * 16, 16)]
        plsc.addupdate_scatter(hist_v, [iv], ones)
        return c
    lax.fori_loop(0, BPW // 16, _hist, 0)
    pltpu.sync_copy(hist_v, hist_hbm.at[wid])


def _k2(idx, emb128):
    mesh = plsc.VectorSubcoreMesh(core_axis_name="c", subcore_axis_name="s")
    f = functools.partial(
        pl.kernel,
        out_type=[
            jax.ShapeDtypeStruct((N_TOK, 128), jnp.float32),
            jax.ShapeDtypeStruct((NW, VOCAB), jnp.float32),
        ],
        mesh=mesh,
        scratch_types=[
            pltpu.VMEM((BPW,), jnp.int32),
            pltpu.VMEM((BPW,), jnp.int32),
            pltpu.VMEM((BPW, 128), jnp.float32),
            pltpu.VMEM((VOCAB,), jnp.float32),
            pltpu.SemaphoreType.DMA,
        ],
        compiler_params=pltpu.CompilerParams(needs_layout_passes=False),
    )(_k2_body)
    return f(idx, emb128)


def _k3_body(z_ref, zqwt_ref, rem_ref, hist_ref,
             o_ref, us_ref, vq_ref, cl_ref, acc_ref):
    i = pl.program_id(0)
    zb = z_ref[0]
    zn = zb / jnp.maximum(jnp.sqrt(jnp.sum(zb * zb, axis=0, keepdims=True)), 1e-12)
    rem = rem_ref[...] & 3
    w = zqwt_ref[...]
    q = w[0:ZC, :]
    for r in range(1, ROWS_PER_WIDE):
        q = jnp.where(rem == r, w[r * ZC:(r + 1) * ZC, :], q)
    qn = q / jnp.maximum(jnp.sqrt(jnp.sum(q * q, axis=0, keepdims=True)), 1e-12)
    diff = qn - zn
    o_ref[0] = zn + diff
    part = jnp.sum(diff * diff)

    @pl.when(i == 0)
    def _():
        acc_ref[0] = part

    @pl.when(i > 0)
    def _():
        acc_ref[0] = acc_ref[0] + part

    @pl.when(i == pl.num_programs(0) - 1)
    def _():
        m = acc_ref[0] / jnp.float32(N_TOK * ZC)
        vq_ref[...] = m.reshape(1, 1)
        cl_ref[...] = (jnp.float32(0.25) * m).reshape(1, 1)
        hit = jnp.sum(hist_ref[...], axis=0)
        used = (hit >= MARGIN).astype(jnp.float32)
        us_ref[...] = (jnp.mean(used) * 100.0).reshape(1, 1)


def _k3(z3, zqwt, idx2, hist):
    return pl.pallas_call(
        _k3_body,
        grid=(8,),
        in_specs=[
            pl.BlockSpec((1, ZC, 1024), lambda i: (i, 0, 0)),
            pl.BlockSpec((128, 1024), lambda i: (0, i)),
            pl.BlockSpec((1, 1024), lambda i: (0, i)),
            pl.BlockSpec((NW, VOCAB), lambda i: (0, 0)),
        ],
        out_specs=[
            pl.BlockSpec((1, ZC, 1024), lambda i: (i, 0, 0)),
            pl.BlockSpec((1, 1), lambda i: (0, 0)),
            pl.BlockSpec((1, 1), lambda i: (0, 0)),
            pl.BlockSpec((1, 1), lambda i: (0, 0)),
        ],
        out_shape=[
            jax.ShapeDtypeStruct((8, ZC, 1024), jnp.float32),
            jax.ShapeDtypeStruct((1, 1), jnp.float32),
            jax.ShapeDtypeStruct((1, 1), jnp.float32),
            jax.ShapeDtypeStruct((1, 1), jnp.float32),
        ],
        scratch_shapes=[pltpu.SMEM((1,), jnp.float32)],
    )(z3, zqwt, idx2, hist)


def kernel(z, embedding):
    z3 = z.reshape(8, ZC, 1024)
    idx = _k1(z3, embedding).reshape(N_TOK)
    emb128 = embedding.reshape(VOCAB // ROWS_PER_WIDE, 128)
    zqw, hist = _k2(idx, emb128)
    zqwt = zqw.T
    zq_out, usage, vq, cl = _k3(z3, zqwt, idx.reshape(1, N_TOK), hist)
    return (zq_out.reshape(8, ZC, 32, 32), usage[0, 0], vq[0, 0], cl[0, 0])


# speedup vs baseline: 1.7261x; 1.0126x over previous
import functools

import jax
import jax.numpy as jnp
from jax import lax
from jax.experimental import pallas as pl
from jax.experimental.pallas import tpu as pltpu
from jax.experimental.pallas import tpu_sc as plsc

VOCAB = 8192
ZC = 32
N_TOK = 8192
TB = 1024
K1_GRID = N_TOK // TB
BLK_PER_BATCH = 1024 // TB
NW = 32
BPW = N_TOK // NW
ROWS_PER_WIDE = 128 // ZC
MARGIN = 1.0 * (N_TOK * ZC / ZC) / VOCAB * 0.08


def _k1_body(z_ref, emb_ref, idx_ref, embn_ref, e2_ref):
    i = pl.program_id(0)

    @pl.when(i == 0)
    def _():
        e = emb_ref[...]
        n = jnp.sqrt(jnp.sum(e * e, axis=1, keepdims=True))
        en = e / jnp.maximum(n, 1e-12)
        embn_ref[...] = (-2.0 * en).astype(jnp.bfloat16)
        e2_ref[...] = jnp.sum(en * en, axis=1, keepdims=True)

    zb = z_ref[0]
    zn = zb / jnp.maximum(jnp.sqrt(jnp.sum(zb * zb, axis=0, keepdims=True)), 1e-12)
    s2 = lax.dot_general(embn_ref[...], zn.astype(jnp.bfloat16),
                         (((1,), (0,)), ((), ())),
                         preferred_element_type=jnp.float32)
    d = e2_ref[...] + s2
    idx_ref[0, 0, :] = jnp.argmin(d, axis=0).astype(jnp.int32)


def _k1(z3, emb):
    return pl.pallas_call(
        _k1_body,
        grid=(K1_GRID,),
        in_specs=[
            pl.BlockSpec((1, ZC, TB),
                         lambda i: (i // BLK_PER_BATCH, 0, i % BLK_PER_BATCH)),
            pl.BlockSpec((VOCAB, ZC), lambda i: (0, 0)),
        ],
        out_specs=pl.BlockSpec((1, 1, TB), lambda i: (i, 0, 0)),
        out_shape=jax.ShapeDtypeStruct((K1_GRID, 1, TB), jnp.int32),
        scratch_shapes=[
            pltpu.VMEM((VOCAB, ZC), jnp.bfloat16),
            pltpu.VMEM((VOCAB, 1), jnp.float32),
        ],
    )(z3, emb)


def _k2_body(idx_hbm, emb_hbm, zqt_hbm, hist_hbm,
             idx_v, ridx_v, rows_v, zqt_v, hist_v, sem):
    wid = lax.axis_index("s") * 2 + lax.axis_index("c")
    base = wid * BPW
    pltpu.sync_copy(idx_hbm.at[pl.ds(base, BPW)], idx_v)

    def _shift(j, c):
        ridx_v[pl.ds(j * 16, 16)] = lax.shift_right_logical(
            idx_v[pl.ds(j * 16, 16)], 2)
        return c
    lax.fori_loop(0, BPW // 16, _shift, 0)

    for k in range(BPW // 128):
        pltpu.async_copy(emb_hbm.at[ridx_v.at[pl.ds(k * 128, 128)]],
                         rows_v.at[pl.ds(k * 128, 128)], sem).wait()

    lanes = lax.iota(jnp.int32, 16)

    def _sel(j, c):
        rowv = lanes + j * 16
        rem = idx_v[pl.ds(j * 16, 16)] & 3
        colbase = rem * 32
        for ch in range(ZC):
            vals = plsc.load_gather(rows_v, [rowv, colbase + ch])
            zqt_v[ch, pl.ds(j * 16, 16)] = vals
        return c
    lax.fori_loop(0, BPW // 16, _sel, 0)
    pltpu.sync_copy(zqt_v, zqt_hbm.at[:, pl.ds(base, BPW)])

    def _zero(j, c):
        hist_v[pl.ds(j * 16, 16)] = jnp.zeros((16,), jnp.float32)
        return c
    lax.fori_loop(0, VOCAB // 16, _zero, 0)
    ones = jnp.ones((16,), jnp.float32)

    def _hist(j, c):
        iv = idx_v[pl.ds(j * 16, 16)]
        plsc.addupdate_scatter(hist_v, [iv], ones)
        return c
    lax.fori_loop(0, BPW // 16, _hist, 0)
    pltpu.sync_copy(hist_v, hist_hbm.at[wid])


def _k2(idx, emb128):
    mesh = plsc.VectorSubcoreMesh(core_axis_name="c", subcore_axis_name="s")
    f = functools.partial(
        pl.kernel,
        out_type=[
            jax.ShapeDtypeStruct((ZC, N_TOK), jnp.float32),
            jax.ShapeDtypeStruct((NW, VOCAB), jnp.float32),
        ],
        mesh=mesh,
        scratch_types=[
            pltpu.VMEM((BPW,), jnp.int32),
            pltpu.VMEM((BPW,), jnp.int32),
            pltpu.VMEM((BPW, 128), jnp.float32),
            pltpu.VMEM((ZC, BPW), jnp.float32),
            pltpu.VMEM((VOCAB,), jnp.float32),
            pltpu.SemaphoreType.DMA,
        ],
        compiler_params=pltpu.CompilerParams(needs_layout_passes=False),
    )(_k2_body)
    return f(idx, emb128)


def _k3_body(z_ref, zqt_ref, hist_ref,
             o_ref, us_ref, vq_ref, cl_ref, acc_ref):
    i = pl.program_id(0)
    zb = z_ref[0]
    zn = zb / jnp.maximum(jnp.sqrt(jnp.sum(zb * zb, axis=0, keepdims=True)), 1e-12)
    q = zqt_ref[...]
    qn = q / jnp.maximum(jnp.sqrt(jnp.sum(q * q, axis=0, keepdims=True)), 1e-12)
    diff = qn - zn
    o_ref[0] = zn + diff
    part = jnp.sum(diff * diff)

    @pl.when(i == 0)
    def _():
        acc_ref[0] = part

    @pl.when(i > 0)
    def _():
        acc_ref[0] = acc_ref[0] + part

    @pl.when(i == pl.num_programs(0) - 1)
    def _():
        m = acc_ref[0] / jnp.float32(N_TOK * ZC)
        vq_ref[...] = m.reshape(1, 1)
        cl_ref[...] = (jnp.float32(0.25) * m).reshape(1, 1)
        hit = jnp.sum(hist_ref[...], axis=0)
        used = (hit >= MARGIN).astype(jnp.float32)
        us_ref[...] = (jnp.mean(used) * 100.0).reshape(1, 1)


def _k3(z3, zqt, hist):
    return pl.pallas_call(
        _k3_body,
        grid=(8,),
        in_specs=[
            pl.BlockSpec((1, ZC, 1024), lambda i: (i, 0, 0)),
            pl.BlockSpec((ZC, 1024), lambda i: (0, i)),
            pl.BlockSpec((NW, VOCAB), lambda i: (0, 0)),
        ],
        out_specs=[
            pl.BlockSpec((1, ZC, 1024), lambda i: (i, 0, 0)),
            pl.BlockSpec((1, 1), lambda i: (0, 0)),
            pl.BlockSpec((1, 1), lambda i: (0, 0)),
            pl.BlockSpec((1, 1), lambda i: (0, 0)),
        ],
        out_shape=[
            jax.ShapeDtypeStruct((8, ZC, 1024), jnp.float32),
            jax.ShapeDtypeStruct((1, 1), jnp.float32),
            jax.ShapeDtypeStruct((1, 1), jnp.float32),
            jax.ShapeDtypeStruct((1, 1), jnp.float32),
        ],
        scratch_shapes=[pltpu.SMEM((1,), jnp.float32)],
    )(z3, zqt, hist)


def kernel(z, embedding):
    z3 = z.reshape(8, ZC, 1024)
    idx = _k1(z3, embedding).reshape(N_TOK)
    emb128 = embedding.reshape(VOCAB // ROWS_PER_WIDE, 128)
    zqt, hist = _k2(idx, emb128)
    zq_out, usage, vq, cl = _k3(z3, zqt, hist)
    return (zq_out.reshape(8, ZC, 32, 32), usage[0, 0], vq[0, 0], cl[0, 0])
